# baseline (speedup 1.0000x reference)
import functools

import jax
import jax.numpy as jnp
from jax import lax
from jax.experimental import pallas as pl
from jax.experimental.pallas import tpu as pltpu
from jax.experimental.pallas import tpu_sc as plsc

N = 100000
E = 6400000
NC = 2
NS = 16
NW = NC * NS
L = 16

NP_ROWS = 6272
NP = NP_ROWS * 16
E_PAD = 6422528
EPT = E_PAD // NW

CH_A = 1024
CHUNKS_A = EPT // CH_A
CH_C = 1024
EPT_S = E_PAD // NS
NCHAIN = 4
CHUNKS_C = EPT_S // CH_C
ZROWS = NP_ROWS // NS

_mesh = plsc.VectorSubcoreMesh(core_axis_name="c", subcore_axis_name="s")
_SC_PARAMS = pltpu.CompilerParams(
    needs_layout_passes=False, use_tc_tiling_on_sc=False)


@functools.partial(
    pl.kernel,
    out_type=(jax.ShapeDtypeStruct((NC, NP_ROWS, 16), jnp.float32),
              jax.ShapeDtypeStruct((NC, NS, NP_ROWS, 16), jnp.float32)),
    mesh=_mesh,
    compiler_params=_SC_PARAMS,
    scratch_types=[
        pltpu.VMEM((NP_ROWS, 16), jnp.float32),
        [pltpu.VMEM((CH_A,), jnp.int32)] * 2,
        pltpu.VMEM((ZROWS, 16), jnp.float32),
        pltpu.VMEM((ZROWS, 16), jnp.float32),
        [pltpu.SemaphoreType.DMA] * 2,
    ],
)
def _deg_kernel(edges, zeros_hbm, out, stage, hist_v, idx_vs, acc_v, tmp_v,
                isems):
    c = lax.axis_index("c")
    s = lax.axis_index("s")
    wid = s * NC + c
    pltpu.sync_copy(zeros_hbm, hist_v)

    ones = jnp.ones((L,), jnp.float32)

    def group(g, _):
        descs = []
        for b in range(2):
            base = wid * EPT + (g * 2 + b) * CH_A
            descs.append(pltpu.async_copy(
                edges.at[0, pl.ds(base, CH_A)], idx_vs[b], isems[b]))
        for b in range(2):
            descs[b].wait()
            for t in range(CH_A // L):
                v = idx_vs[b][pl.ds(t * L, L)]
                row = lax.shift_right_logical(v, 4)
                col = lax.bitwise_and(v, 15)
                plsc.addupdate_scatter(hist_v, [row, col], ones)
        return 0

    lax.fori_loop(0, CHUNKS_A // 2, group, 0)

    pltpu.sync_copy(hist_v, stage.at[c, s])
    plsc.subcore_barrier()

    def vinit(i, _):
        acc_v[i, :] = hist_v[s * ZROWS + i, :]
        return 0

    lax.fori_loop(0, ZROWS, vinit, 0)
    for t in range(NS - 1):
        other = lax.rem(s + 1 + t, NS)
        pltpu.sync_copy(stage.at[c, other, pl.ds(s * ZROWS, ZROWS)], tmp_v)

        def vadd(i, _):
            acc_v[i, :] = acc_v[i, :] + tmp_v[i, :]
            return 0

        lax.fori_loop(0, ZROWS, vadd, 0)
    pltpu.sync_copy(acc_v, out.at[c, pl.ds(s * ZROWS, ZROWS)])


NP_AGG = 100016
ZROWS_AGG = NP_AGG // NS


@functools.partial(
    pl.kernel,
    out_type=jax.ShapeDtypeStruct((NC, NP_AGG, 8), jnp.float32),
    mesh=_mesh,
    compiler_params=_SC_PARAMS,
    scratch_types=[
        pltpu.VMEM((CH_C, 8), jnp.float32),
        pltpu.VMEM((CH_C,), jnp.int32),
        pltpu.VMEM((CH_C,), jnp.int32),
        pltpu.VMEM_SHARED((NP_AGG, 8), jnp.float32),
        pltpu.VMEM_SHARED((NP_AGG, 8), jnp.float32),
        pltpu.SemaphoreType.DMA,
    ],
)
def _agg_kernel(feat, edges, zeros_hbm, out, rows_v, src_v, dst_v, feat_sh,
                agg_sh, sem):
    c = lax.axis_index("c")
    s = lax.axis_index("s")
    pltpu.sync_copy(feat.at[c, pl.ds(s * ZROWS_AGG, ZROWS_AGG)],
                    feat_sh.at[pl.ds(s * ZROWS_AGG, ZROWS_AGG)])
    pltpu.sync_copy(zeros_hbm, agg_sh.at[pl.ds(s * ZROWS_AGG, ZROWS_AGG)])
    plsc.subcore_barrier()

    def chunk(i, _):
        base = s * EPT_S + i * CH_C
        d0 = pltpu.async_copy(edges.at[0, pl.ds(base, CH_C)], src_v, sem)
        d1 = pltpu.async_copy(edges.at[1, pl.ds(base, CH_C)], dst_v, sem)
        d0.wait()
        d1.wait()
        pltpu.async_copy(feat_sh.at[src_v], rows_v, sem).wait()
        pltpu.async_copy(rows_v, agg_sh.at[dst_v], sem, add=True).wait()
        return 0

    lax.fori_loop(0, CHUNKS_C, chunk, 0)

    plsc.subcore_barrier()
    pltpu.sync_copy(agg_sh.at[pl.ds(s * ZROWS_AGG, ZROWS_AGG)],
                    out.at[c, pl.ds(s * ZROWS_AGG, ZROWS_AGG)])


R_B = 6272


def _feat_body(x_ref, deg_ref, feat_ref):
    i = pl.program_id(0)
    xs = x_ref[:, 15:25]
    deg = deg_ref[0] + deg_ref[1]
    norm = jnp.where(deg > 0.0, lax.rsqrt(deg), 0.0)
    rows = i * R_B + lax.broadcasted_iota(jnp.int32, (R_B, 1), 0)
    real = rows < N
    val = jnp.where(real, xs * norm, 0.0)
    cnt = jnp.where(real, 1.0, 0.0)
    lo = val[:, 0:8]
    hi = jnp.concatenate(
        [val[:, 8:10], cnt, jnp.zeros((R_B, 5), jnp.float32)], axis=1)
    feat_ref[...] = jnp.stack([lo, hi])


_feat_kernel = pl.pallas_call(
    _feat_body,
    grid=(NP // R_B,),
    in_specs=[
        pl.BlockSpec((R_B, 128), lambda i: (i, 0)),
        pl.BlockSpec((NC, R_B, 1), lambda i: (0, i, 0)),
    ],
    out_specs=pl.BlockSpec((NC, R_B, 8), lambda i: (0, i, 0)),
    out_shape=jax.ShapeDtypeStruct((NC, NP, 8), jnp.float32),
)


R_D = 5000


def _proj_body(p_ref, w1_ref, b1_ref, w2_ref, b2_ref, out_ref):
    agg = jnp.concatenate([p_ref[0], p_ref[1][:, 0:2]], axis=1)
    deg = p_ref[1][:, 2:3]
    norm = jnp.where(deg > 0.0, lax.rsqrt(deg), 0.0)
    h = jnp.dot(agg * norm, w1_ref[...],
                preferred_element_type=jnp.float32) + b1_ref[...]
    out_ref[...] = jnp.dot(h, w2_ref[...],
                           preferred_element_type=jnp.float32) + b2_ref[...]


_proj_kernel = pl.pallas_call(
    _proj_body,
    grid=(N // R_D,),
    in_specs=[
        pl.BlockSpec((NC, R_D, 8), lambda i: (0, i, 0)),
        pl.BlockSpec((10, 16), lambda i: (0, 0)),
        pl.BlockSpec((1, 16), lambda i: (0, 0)),
        pl.BlockSpec((16, 16), lambda i: (0, 0)),
        pl.BlockSpec((1, 16), lambda i: (0, 0)),
    ],
    out_specs=pl.BlockSpec((R_D, 16), lambda i: (i, 0)),
    out_shape=jax.ShapeDtypeStruct((N, 16), jnp.float32),
)


def kernel(x, edge_index, W1, b1, W2, b2):
    e = edge_index.astype(jnp.int32)
    pad = jnp.full((2, E_PAD - E), N, jnp.int32)
    edges = jnp.concatenate([e, pad], axis=1)
    zeros2d = jnp.zeros((NP_ROWS, 16), jnp.float32)
    zeros_agg = jnp.zeros((ZROWS_AGG, 8), jnp.float32)

    deg, _ = _deg_kernel(edges, zeros2d)
    out_deg = deg.reshape(NC, NP, 1)

    feat = _feat_kernel(x, out_deg)[:, :NP_AGG]
    partials = _agg_kernel(feat, edges, zeros_agg)

    return _proj_kernel(partials, W1,
                        b1.reshape(1, 16), W2, b2.reshape(1, 16))


# speedup vs baseline: 1.1614x; 1.1614x over previous
import functools

import jax
import jax.numpy as jnp
from jax import lax
from jax.experimental import pallas as pl
from jax.experimental.pallas import tpu as pltpu
from jax.experimental.pallas import tpu_sc as plsc

N = 100000
E = 6400000
NC = 2
NS = 16
NW = NC * NS
L = 16

NP_ROWS = 6272
NP = NP_ROWS * 16
EPT = E // NW

CH_A = 800
CHUNKS_A = EPT // CH_A
CH_C = 1000
CHUNKS_C = EPT // CH_C
ZROWS = NP_ROWS // NS

_mesh = plsc.VectorSubcoreMesh(core_axis_name="c", subcore_axis_name="s")
_SC_PARAMS = pltpu.CompilerParams(
    needs_layout_passes=False, use_tc_tiling_on_sc=False)


@functools.partial(
    pl.kernel,
    out_type=(jax.ShapeDtypeStruct((NC, NP_ROWS, 16), jnp.float32),
              jax.ShapeDtypeStruct((NC, NS, NP_ROWS, 16), jnp.float32)),
    mesh=_mesh,
    compiler_params=_SC_PARAMS,
    scratch_types=[
        pltpu.VMEM((NP_ROWS, 16), jnp.float32),
        [pltpu.VMEM((CH_A,), jnp.int32)] * 2,
        pltpu.VMEM((ZROWS, 16), jnp.float32),
        pltpu.VMEM((ZROWS, 16), jnp.float32),
        [pltpu.SemaphoreType.DMA] * 2,
    ],
)
def _deg_kernel(edges, zeros_hbm, out, stage, hist_v, idx_vs, acc_v, tmp_v,
                isems):
    c = lax.axis_index("c")
    s = lax.axis_index("s")
    wid = s * NC + c
    pltpu.sync_copy(zeros_hbm, hist_v)

    ones = jnp.ones((L,), jnp.float32)

    def group(g, _):
        descs = []
        for b in range(2):
            base = wid * EPT + (g * 2 + b) * CH_A
            descs.append(pltpu.async_copy(
                edges.at[0, pl.ds(base, CH_A)], idx_vs[b], isems[b]))
        for b in range(2):
            descs[b].wait()
            for t in range(CH_A // L):
                v = idx_vs[b][pl.ds(t * L, L)]
                row = lax.shift_right_logical(v, 4)
                col = lax.bitwise_and(v, 15)
                plsc.addupdate_scatter(hist_v, [row, col], ones)
        return 0

    lax.fori_loop(0, CHUNKS_A // 2, group, 0)

    pltpu.sync_copy(hist_v, stage.at[c, s])
    plsc.subcore_barrier()

    def vinit(i, _):
        acc_v[i, :] = hist_v[s * ZROWS + i, :]
        return 0

    lax.fori_loop(0, ZROWS, vinit, 0)
    for t in range(NS - 1):
        other = lax.rem(s + 1 + t, NS)
        pltpu.sync_copy(stage.at[c, other, pl.ds(s * ZROWS, ZROWS)], tmp_v)

        def vadd(i, _):
            acc_v[i, :] = acc_v[i, :] + tmp_v[i, :]
            return 0

        lax.fori_loop(0, ZROWS, vadd, 0)
    pltpu.sync_copy(acc_v, out.at[c, pl.ds(s * ZROWS, ZROWS)])


NP_AGG = 100016
ZROWS_AGG = NP_AGG // NS


@functools.partial(
    pl.kernel,
    out_type=jax.ShapeDtypeStruct((NC, NP_AGG, 16), jnp.float32),
    mesh=_mesh,
    compiler_params=_SC_PARAMS,
    scratch_types=[
        pltpu.VMEM((CH_C, 16), jnp.float32),
        pltpu.VMEM((CH_C,), jnp.int32),
        pltpu.VMEM((CH_C,), jnp.int32),
        pltpu.VMEM_SHARED((NP_AGG, 16), jnp.float32),
        pltpu.SemaphoreType.DMA,
    ],
)
def _agg_kernel(feat, edges, zeros_hbm, out, rows_v, src_v, dst_v, agg_sh,
                sem):
    c = lax.axis_index("c")
    s = lax.axis_index("s")
    wid = s * NC + c
    pltpu.sync_copy(zeros_hbm, agg_sh.at[pl.ds(s * ZROWS_AGG, ZROWS_AGG)])
    plsc.subcore_barrier()

    def chunk(i, _):
        base = wid * EPT + i * CH_C
        d0 = pltpu.async_copy(edges.at[0, pl.ds(base, CH_C)], src_v, sem)
        d1 = pltpu.async_copy(edges.at[1, pl.ds(base, CH_C)], dst_v, sem)
        d0.wait()
        d1.wait()
        pltpu.async_copy(feat.at[src_v], rows_v, sem).wait()
        pltpu.async_copy(rows_v, agg_sh.at[dst_v], sem, add=True).wait()
        return 0

    lax.fori_loop(0, CHUNKS_C, chunk, 0)

    plsc.subcore_barrier()
    pltpu.sync_copy(agg_sh.at[pl.ds(s * ZROWS_AGG, ZROWS_AGG)],
                    out.at[c, pl.ds(s * ZROWS_AGG, ZROWS_AGG)])


R_B = 6272


def _feat_body(x_ref, deg_ref, feat_ref):
    i = pl.program_id(0)
    xs = x_ref[:, 15:25]
    deg = deg_ref[0] + deg_ref[1]
    norm = jnp.where(deg > 0.0, lax.rsqrt(deg), 0.0)
    rows = i * R_B + lax.broadcasted_iota(jnp.int32, (R_B, 1), 0)
    real = rows < N
    val = jnp.where(real, xs * norm, 0.0)
    cnt = jnp.where(real, 1.0, 0.0)
    feat_ref[...] = jnp.concatenate(
        [val, cnt, jnp.zeros((R_B, 5), jnp.float32)], axis=1)


_feat_kernel = pl.pallas_call(
    _feat_body,
    grid=(NP // R_B,),
    in_specs=[
        pl.BlockSpec((R_B, 128), lambda i: (i, 0)),
        pl.BlockSpec((NC, R_B, 1), lambda i: (0, i, 0)),
    ],
    out_specs=pl.BlockSpec((R_B, 16), lambda i: (i, 0)),
    out_shape=jax.ShapeDtypeStruct((NP, 16), jnp.float32),
)


R_D = 5000


def _proj_body(p_ref, w1_ref, b1_ref, w2_ref, b2_ref, out_ref):
    agg = p_ref[0] + p_ref[1]
    deg = agg[:, 10:11]
    norm = jnp.where(deg > 0.0, lax.rsqrt(deg), 0.0)
    h = jnp.dot(agg * norm, w1_ref[...],
                preferred_element_type=jnp.float32) + b1_ref[...]
    out_ref[...] = jnp.dot(h, w2_ref[...],
                           preferred_element_type=jnp.float32) + b2_ref[...]


_proj_kernel = pl.pallas_call(
    _proj_body,
    grid=(N // R_D,),
    in_specs=[
        pl.BlockSpec((NC, R_D, 16), lambda i: (0, i, 0)),
        pl.BlockSpec((16, 16), lambda i: (0, 0)),
        pl.BlockSpec((1, 16), lambda i: (0, 0)),
        pl.BlockSpec((16, 16), lambda i: (0, 0)),
        pl.BlockSpec((1, 16), lambda i: (0, 0)),
    ],
    out_specs=pl.BlockSpec((R_D, 16), lambda i: (i, 0)),
    out_shape=jax.ShapeDtypeStruct((N, 16), jnp.float32),
)


def kernel(x, edge_index, W1, b1, W2, b2):
    edges = edge_index.astype(jnp.int32)
    zeros2d = jnp.zeros((NP_ROWS, 16), jnp.float32)
    zeros_agg = jnp.zeros((ZROWS_AGG, 16), jnp.float32)

    deg, _ = _deg_kernel(edges, zeros2d)
    out_deg = deg.reshape(NC, NP, 1)

    feat = _feat_kernel(x, out_deg)
    partials = _agg_kernel(feat, edges, zeros_agg)

    w1p = jnp.zeros((16, 16), jnp.float32).at[:10].set(W1)
    return _proj_kernel(partials, w1p,
                        b1.reshape(1, 16), W2, b2.reshape(1, 16))
